# baseline (speedup 1.0000x reference)
import functools

import jax
import jax.numpy as jnp
from jax.experimental import pallas as pl
from jax.experimental.pallas import tpu as pltpu

_LANE = 128


def _mlp_kernel(x0_ref, x1_ref, p_ref, q_ref, r_ref, lo_ref, hi_ref, b2_ref,
                o_ref, *, hidden):
    o_ref[:] = jnp.full(o_ref.shape, b2_ref[0], dtype=jnp.float32)

    group = 50
    assert hidden % group == 0

    @pl.loop(0, hidden // group)
    def _(g):
        x0 = x0_ref[:]
        x1 = x1_ref[:]
        c = None
        for k in range(group):
            j = g * group + k
            u = p_ref[j] * x0 + (q_ref[j] * x1 + r_ref[j])
            t = jnp.minimum(jnp.maximum(u, lo_ref[j]), hi_ref[j])
            c = t if c is None else c + t
        o_ref[:] = o_ref[:] + c

def kernel(x, w1, b1, w2, b2, *, tile_rows=1024, strip_rows=64):
    B, F = x.shape
    assert F == 2
    H = w1.shape[0]
    strip = int(strip_rows)

    chunk = strip * _LANE
    b_pad = pl.cdiv(B, chunk) * chunk
    rows_pad = b_pad // _LANE
    pad = b_pad - B

    c0 = x[:, 0]
    c1 = x[:, 1]
    if pad:
        c0 = jnp.pad(c0, (0, pad))
        c1 = jnp.pad(c1, (0, pad))
    x0 = c0.reshape(rows_pad, _LANE)
    x1 = c1.reshape(rows_pad, _LANE)

    w2f = jnp.asarray(w2, jnp.float32).reshape(H)
    p = w2f * jnp.asarray(w1[:, 0], jnp.float32)
    q = w2f * jnp.asarray(w1[:, 1], jnp.float32)
    r = w2f * jnp.asarray(b1, jnp.float32)
    big = jnp.float32(3.0e38)
    pos = w2f >= 0
    lo = jnp.where(pos, jnp.float32(0), -big)
    hi = jnp.where(pos, big, jnp.float32(0))
    b2f = jnp.asarray(b2, jnp.float32).reshape(1)

    tr = min(int(tile_rows), rows_pad)
    tr = max(strip, (tr // strip) * strip)
    num_blocks = pl.cdiv(rows_pad, tr)

    smem = pl.BlockSpec(memory_space=pltpu.MemorySpace.SMEM)
    out = pl.pallas_call(
        functools.partial(_mlp_kernel, hidden=H),
        out_shape=jax.ShapeDtypeStruct((rows_pad, _LANE), jnp.float32),
        grid_spec=pltpu.PrefetchScalarGridSpec(
            num_scalar_prefetch=0,
            grid=(num_blocks,),
            in_specs=[
                pl.BlockSpec((tr, _LANE), lambda i: (i, 0)),
                pl.BlockSpec((tr, _LANE), lambda i: (i, 0)),
                smem, smem, smem, smem, smem, smem,
            ],
            out_specs=pl.BlockSpec((tr, _LANE), lambda i: (i, 0)),
        ),
        compiler_params=pltpu.CompilerParams(
            dimension_semantics=("parallel",),
            vmem_limit_bytes=64 * 1024 * 1024,
        ),
    )(x0, x1, p, q, r, lo, hi, b2f)

    return out.reshape(rows_pad * _LANE)[:B].reshape(B, 1)


# speedup vs baseline: 1.0025x; 1.0025x over previous
import functools

import jax
import jax.numpy as jnp
from jax.experimental import pallas as pl
from jax.experimental.pallas import tpu as pltpu

_LANE = 128


def _mlp_kernel(x0_ref, x1_ref, p_ref, q_ref, r_ref, lo_ref, hi_ref, b2_ref,
                o_ref, *, hidden):
    o_ref[:] = jnp.full(o_ref.shape, b2_ref[0], dtype=jnp.float32)

    group = 50
    assert hidden % group == 0

    @pl.loop(0, hidden // group)
    def _(g):
        x0 = x0_ref[:]
        x1 = x1_ref[:]
        c = None
        for k in range(group):
            j = g * group + k
            u = p_ref[j] * x0 + (q_ref[j] * x1 + r_ref[j])
            t = jnp.minimum(jnp.maximum(u, lo_ref[j]), hi_ref[j])
            c = t if c is None else c + t
        o_ref[:] = o_ref[:] + c

def kernel(x, w1, b1, w2, b2, *, tile_rows=2048, strip_rows=64):
    B, F = x.shape
    assert F == 2
    H = w1.shape[0]
    strip = int(strip_rows)

    chunk = strip * _LANE
    b_pad = pl.cdiv(B, chunk) * chunk
    rows_pad = b_pad // _LANE
    pad = b_pad - B

    c0 = x[:, 0]
    c1 = x[:, 1]
    if pad:
        c0 = jnp.pad(c0, (0, pad))
        c1 = jnp.pad(c1, (0, pad))
    x0 = c0.reshape(rows_pad, _LANE)
    x1 = c1.reshape(rows_pad, _LANE)

    w2f = jnp.asarray(w2, jnp.float32).reshape(H)
    p = w2f * jnp.asarray(w1[:, 0], jnp.float32)
    q = w2f * jnp.asarray(w1[:, 1], jnp.float32)
    r = w2f * jnp.asarray(b1, jnp.float32)
    big = jnp.float32(3.0e38)
    pos = w2f >= 0
    lo = jnp.where(pos, jnp.float32(0), -big)
    hi = jnp.where(pos, big, jnp.float32(0))
    b2f = jnp.asarray(b2, jnp.float32).reshape(1)

    tr = min(int(tile_rows), rows_pad)
    tr = max(strip, (tr // strip) * strip)
    num_blocks = pl.cdiv(rows_pad, tr)

    smem = pl.BlockSpec(memory_space=pltpu.MemorySpace.SMEM)
    out = pl.pallas_call(
        functools.partial(_mlp_kernel, hidden=H),
        out_shape=jax.ShapeDtypeStruct((rows_pad, _LANE), jnp.float32),
        grid_spec=pltpu.PrefetchScalarGridSpec(
            num_scalar_prefetch=0,
            grid=(num_blocks,),
            in_specs=[
                pl.BlockSpec((tr, _LANE), lambda i: (i, 0)),
                pl.BlockSpec((tr, _LANE), lambda i: (i, 0)),
                smem, smem, smem, smem, smem, smem,
            ],
            out_specs=pl.BlockSpec((tr, _LANE), lambda i: (i, 0)),
        ),
        compiler_params=pltpu.CompilerParams(
            dimension_semantics=("parallel",),
            vmem_limit_bytes=64 * 1024 * 1024,
        ),
    )(x0, x1, p, q, r, lo, hi, b2f)

    return out.reshape(rows_pad * _LANE)[:B].reshape(B, 1)
